# baseline (speedup 1.0000x reference)
import functools

import jax
import jax.numpy as jnp
from jax.experimental import pallas as pl
from jax.experimental.pallas import tpu as pltpu

UP_RATIO = 10
K = 5
STD_FACTOR = 3.0

_SELF_TQ = 256
_MAIN_TQ = 512


def _coords(ref):
    return ref[0, :, 0:1], ref[0, :, 1:2], ref[0, :, 2:3]


def _rows(ref):
    return ref[0, 0:1, :], ref[0, 1:2, :], ref[0, 2:3, :]


def _bf16(x):
    return x.astype(jnp.bfloat16).astype(jnp.float32)


def _sel_and_exact_d2(q_ref, pt_ref):
    qx, qy, qz = _coords(q_ref)
    px, py, pz = _rows(pt_ref)
    q2 = qx * qx + qy * qy + qz * qz
    p2 = px * px + py * py + pz * pz
    qb = q_ref[0].astype(jnp.bfloat16)
    pb2 = (-2.0 * pt_ref[0].astype(jnp.bfloat16).astype(jnp.float32)
           ).astype(jnp.bfloat16)
    qp2 = jnp.dot(qb, pb2, preferred_element_type=jnp.float32)
    d2_sel = (q2 + p2) + qp2
    dx = qx - px
    dy = qy - py
    dz = qz - pz
    d2_exact = dx * dx + dy * dy + dz * dz
    return d2_sel, d2_exact


def _self_knn_body(q_ref, pt_ref, out_ref, *, n):
    d2_sel, d2_exact = _sel_and_exact_d2(q_ref, pt_ref)
    for rank in range(2):
        m = jnp.min(d2_sel, axis=1, keepdims=True)
        g = jnp.where(d2_sel <= m, 1.0, 0.0)
        if rank == 1:
            out_ref[0] = jnp.sum(g * d2_exact, axis=1, keepdims=True)
        else:
            d2_sel = d2_sel + g * 1e30


def _main_body(q_ref, ptt_ref, pts_ref, pmh_ref, pml_ref, out_ref, *, n):
    tq = q_ref.shape[1]

    def top5(pt_ref, invs):
        d2_sel, d2_exact = _sel_and_exact_d2(q_ref, pt_ref)
        dists = []
        u_raw = jnp.zeros((tq, n), jnp.float32)
        for k in range(K):
            m = jnp.min(d2_sel, axis=1, keepdims=True)
            onehot = d2_sel <= m
            dk = jnp.sum(jnp.where(onehot, d2_exact, 0.0),
                         axis=1, keepdims=True)
            dists.append(dk)
            wk = 1.0 / (dk + 1e-8) if invs is None else invs[k]
            u_raw = u_raw + jnp.where(onehot, wk, 0.0)
            if k < K - 1:
                d2_sel = jnp.where(onehot, jnp.inf, d2_sel)
        return dists, u_raw

    mt, u_t = top5(ptt_ref, None)
    inv = [1.0 / (m + 1e-8) for m in mt]
    norm = inv[0] + inv[1] + inv[2] + inv[3] + inv[4]
    rnorm = 1.0 / norm

    ms, u_s = top5(pts_ref, inv)

    udf_t = jnp.zeros((tq, 1), jnp.float32)
    udf_s = jnp.zeros((tq, 1), jnp.float32)
    for k in range(K):
        udf_t = udf_t + jnp.sqrt(mt[k] + 1e-10) * inv[k]
        udf_s = udf_s + jnp.sqrt(ms[k] + 1e-10) * inv[k]

    u = jnp.concatenate([u_t * rnorm, u_s * (-rnorm)],
                        axis=1).astype(jnp.bfloat16)
    wpd = (jnp.dot(u, pmh_ref[0], preferred_element_type=jnp.float32)
           + jnp.dot(u, pml_ref[0], preferred_element_type=jnp.float32))
    gd = jnp.sum(jnp.abs(wpd), axis=1, keepdims=True)
    err = jnp.abs(udf_t - udf_s) * rnorm + gd
    out_ref[0] = err


@jax.jit
def kernel(src, tgt):
    b, n, _ = tgt.shape
    nq = n * UP_RATIO + src.shape[1]

    tgt_t = jnp.swapaxes(tgt, 1, 2)
    src_t = jnp.swapaxes(src, 1, 2)

    self_d2 = pl.pallas_call(
        functools.partial(_self_knn_body, n=n),
        grid=(b, n // _SELF_TQ),
        in_specs=[
            pl.BlockSpec((1, _SELF_TQ, 3), lambda i, j: (i, j, 0)),
            pl.BlockSpec((1, 3, n), lambda i, j: (i, 0, 0)),
        ],
        out_specs=pl.BlockSpec((1, _SELF_TQ, 1), lambda i, j: (i, j, 0)),
        out_shape=jax.ShapeDtypeStruct((b, n, 1), jnp.float32),
    )(tgt, tgt_t)

    std = jnp.sqrt(self_d2 + 1e-10) * STD_FACTOR
    noise = jax.random.normal(
        jax.random.key(42), (b, n, UP_RATIO, 3), dtype=jnp.float32
    ) * std[..., None]
    query = (tgt[:, :, None, :] + noise).reshape(b, -1, 3)
    query = jnp.concatenate([query, src], axis=1)

    pad = jnp.zeros((b, n, 5), jnp.float32)
    tgt_pad = jnp.concatenate([tgt, pad], axis=2)
    src_pad = jnp.concatenate([src, pad], axis=2)
    pm = jnp.concatenate([tgt_pad, src_pad], axis=1)
    pm_hi = pm.astype(jnp.bfloat16)
    pm_lo = (pm - pm_hi.astype(jnp.float32)).astype(jnp.bfloat16)

    err = pl.pallas_call(
        functools.partial(_main_body, n=n),
        grid=(b, nq // _MAIN_TQ),
        in_specs=[
            pl.BlockSpec((1, _MAIN_TQ, 3), lambda i, j: (i, j, 0)),
            pl.BlockSpec((1, 3, n), lambda i, j: (i, 0, 0)),
            pl.BlockSpec((1, 3, n), lambda i, j: (i, 0, 0)),
            pl.BlockSpec((1, 2 * n, 8), lambda i, j: (i, 0, 0)),
            pl.BlockSpec((1, 2 * n, 8), lambda i, j: (i, 0, 0)),
        ],
        out_specs=pl.BlockSpec((1, _MAIN_TQ, 1), lambda i, j: (i, j, 0)),
        out_shape=jax.ShapeDtypeStruct((b, nq, 1), jnp.float32),
        compiler_params=pltpu.CompilerParams(
            dimension_semantics=("parallel", "parallel")),
    )(query, tgt_t, src_t, pm_hi, pm_lo)

    return jnp.sum(err) / b / nq


# speedup vs baseline: 1.0097x; 1.0097x over previous
import functools

import jax
import jax.numpy as jnp
from jax.experimental import pallas as pl
from jax.experimental.pallas import tpu as pltpu

UP_RATIO = 10
K = 5
STD_FACTOR = 3.0

_SELF_TQ = 256
_MAIN_TQ = 512

_NOISE_CACHE = {}


def _noise_table(b, n):
    key = (b, n)
    if key not in _NOISE_CACHE:
        with jax.ensure_compile_time_eval():
            _NOISE_CACHE[key] = jax.random.normal(
                jax.random.key(42), (b, n, UP_RATIO, 3), dtype=jnp.float32)
    return _NOISE_CACHE[key]


def _coords(ref):
    return ref[0, :, 0:1], ref[0, :, 1:2], ref[0, :, 2:3]


def _rows(ref):
    return ref[0, 0:1, :], ref[0, 1:2, :], ref[0, 2:3, :]


def _bf16(x):
    return x.astype(jnp.bfloat16).astype(jnp.float32)


def _sel_and_exact_d2(q_ref, pt_ref):
    qx, qy, qz = _coords(q_ref)
    px, py, pz = _rows(pt_ref)
    q2 = qx * qx + qy * qy + qz * qz
    p2 = px * px + py * py + pz * pz
    qb = q_ref[0].astype(jnp.bfloat16)
    pb2 = (-2.0 * pt_ref[0].astype(jnp.bfloat16).astype(jnp.float32)
           ).astype(jnp.bfloat16)
    qp2 = jnp.dot(qb, pb2, preferred_element_type=jnp.float32)
    d2_sel = (q2 + p2) + qp2
    dx = qx - px
    dy = qy - py
    dz = qz - pz
    d2_exact = dx * dx + dy * dy + dz * dz
    return d2_sel, d2_exact


def _self_knn_body(q_ref, pt_ref, out_ref, *, n):
    d2_sel, d2_exact = _sel_and_exact_d2(q_ref, pt_ref)
    for rank in range(2):
        m = jnp.min(d2_sel, axis=1, keepdims=True)
        g = jnp.where(d2_sel <= m, 1.0, 0.0)
        if rank == 1:
            out_ref[0] = jnp.sum(g * d2_exact, axis=1, keepdims=True)
        else:
            d2_sel = d2_sel + g * 1e30


def _main_body(q_ref, ptt_ref, pts_ref, pmh_ref, pml_ref, out_ref, *, n):
    tq = q_ref.shape[1]

    def top5(pt_ref, invs):
        d2_sel, d2_exact = _sel_and_exact_d2(q_ref, pt_ref)
        dists = []
        u_raw = jnp.zeros((tq, n), jnp.float32)
        for k in range(K):
            m = jnp.min(d2_sel, axis=1, keepdims=True)
            onehot = d2_sel <= m
            dk = jnp.sum(jnp.where(onehot, d2_exact, 0.0),
                         axis=1, keepdims=True)
            dists.append(dk)
            wk = 1.0 / (dk + 1e-8) if invs is None else invs[k]
            u_raw = u_raw + jnp.where(onehot, wk, 0.0)
            if k < K - 1:
                d2_sel = jnp.where(onehot, jnp.inf, d2_sel)
        return dists, u_raw

    mt, u_t = top5(ptt_ref, None)
    inv = [1.0 / (m + 1e-8) for m in mt]
    norm = inv[0] + inv[1] + inv[2] + inv[3] + inv[4]
    rnorm = 1.0 / norm

    ms, u_s = top5(pts_ref, inv)

    udf_t = jnp.zeros((tq, 1), jnp.float32)
    udf_s = jnp.zeros((tq, 1), jnp.float32)
    for k in range(K):
        udf_t = udf_t + jnp.sqrt(mt[k] + 1e-10) * inv[k]
        udf_s = udf_s + jnp.sqrt(ms[k] + 1e-10) * inv[k]

    u = jnp.concatenate([u_t * rnorm, u_s * (-rnorm)],
                        axis=1).astype(jnp.bfloat16)
    wpd = (jnp.dot(u, pmh_ref[0], preferred_element_type=jnp.float32)
           + jnp.dot(u, pml_ref[0], preferred_element_type=jnp.float32))
    gd = jnp.sum(jnp.abs(wpd), axis=1, keepdims=True)
    err = jnp.abs(udf_t - udf_s) * rnorm + gd
    out_ref[0] = err


@jax.jit
def kernel(src, tgt):
    b, n, _ = tgt.shape
    nq = n * UP_RATIO + src.shape[1]

    tgt_t = jnp.swapaxes(tgt, 1, 2)
    src_t = jnp.swapaxes(src, 1, 2)

    self_d2 = pl.pallas_call(
        functools.partial(_self_knn_body, n=n),
        grid=(b, n // _SELF_TQ),
        in_specs=[
            pl.BlockSpec((1, _SELF_TQ, 3), lambda i, j: (i, j, 0)),
            pl.BlockSpec((1, 3, n), lambda i, j: (i, 0, 0)),
        ],
        out_specs=pl.BlockSpec((1, _SELF_TQ, 1), lambda i, j: (i, j, 0)),
        out_shape=jax.ShapeDtypeStruct((b, n, 1), jnp.float32),
    )(tgt, tgt_t)

    std = jnp.sqrt(self_d2 + 1e-10) * STD_FACTOR
    noise = _noise_table(b, n) * std[..., None]
    query = (tgt[:, :, None, :] + noise).reshape(b, -1, 3)
    query = jnp.concatenate([query, src], axis=1)

    pad = jnp.zeros((b, n, 5), jnp.float32)
    tgt_pad = jnp.concatenate([tgt, pad], axis=2)
    src_pad = jnp.concatenate([src, pad], axis=2)
    pm = jnp.concatenate([tgt_pad, src_pad], axis=1)
    pm_hi = pm.astype(jnp.bfloat16)
    pm_lo = (pm - pm_hi.astype(jnp.float32)).astype(jnp.bfloat16)

    err = pl.pallas_call(
        functools.partial(_main_body, n=n),
        grid=(b, nq // _MAIN_TQ),
        in_specs=[
            pl.BlockSpec((1, _MAIN_TQ, 3), lambda i, j: (i, j, 0)),
            pl.BlockSpec((1, 3, n), lambda i, j: (i, 0, 0)),
            pl.BlockSpec((1, 3, n), lambda i, j: (i, 0, 0)),
            pl.BlockSpec((1, 2 * n, 8), lambda i, j: (i, 0, 0)),
            pl.BlockSpec((1, 2 * n, 8), lambda i, j: (i, 0, 0)),
        ],
        out_specs=pl.BlockSpec((1, _MAIN_TQ, 1), lambda i, j: (i, j, 0)),
        out_shape=jax.ShapeDtypeStruct((b, nq, 1), jnp.float32),
        compiler_params=pltpu.CompilerParams(
            dimension_semantics=("parallel", "parallel")),
    )(query, tgt_t, src_t, pm_hi, pm_lo)

    return jnp.sum(err) / b / nq


# speedup vs baseline: 1.0136x; 1.0039x over previous
import functools

import jax
import jax.numpy as jnp
from jax.experimental import pallas as pl
from jax.experimental.pallas import tpu as pltpu

UP_RATIO = 10
K = 5
STD_FACTOR = 3.0

_TQ = 512

_NOISE_CACHE = {}


def _noise_table(b, n):
    key = (b, n)
    if key not in _NOISE_CACHE:
        with jax.ensure_compile_time_eval():
            nz = jax.random.normal(
                jax.random.key(42), (b, n, UP_RATIO, 3), dtype=jnp.float32)
            _NOISE_CACHE[key] = jnp.swapaxes(nz, 1, 2)
    return _NOISE_CACHE[key]


def _sel_and_exact_d2(qv, pt_ref):
    qx = qv[:, 0:1]
    qy = qv[:, 1:2]
    qz = qv[:, 2:3]
    px = pt_ref[0, 0:1, :]
    py = pt_ref[0, 1:2, :]
    pz = pt_ref[0, 2:3, :]
    q2 = qx * qx + qy * qy + qz * qz
    p2 = px * px + py * py + pz * pz
    qb = qv.astype(jnp.bfloat16)
    pb2 = (-2.0 * pt_ref[0].astype(jnp.bfloat16).astype(jnp.float32)
           ).astype(jnp.bfloat16)
    qp2 = jnp.dot(qb, pb2, preferred_element_type=jnp.float32)
    d2_sel = (q2 + p2) + qp2
    dx = qx - px
    dy = qy - py
    dz = qz - pz
    d2_exact = dx * dx + dy * dy + dz * dz
    return d2_sel, d2_exact


def _fused_body(tgt_blk_ref, src_blk_ref, noise_ref, ptt_ref, pts_ref,
                pmh_ref, pml_ref, out_ref, std_ref, *, n, n_self, n_noisy):
    j = pl.program_id(1)
    tq = tgt_blk_ref.shape[1]

    @pl.when(j < n_self)
    def _self_phase():
        qv = tgt_blk_ref[0]
        d2_sel, d2_exact = _sel_and_exact_d2(qv, ptt_ref)
        for rank in range(2):
            m = jnp.min(d2_sel, axis=1, keepdims=True)
            onehot = d2_sel <= m
            if rank == 1:
                d2nd = jnp.sum(jnp.where(onehot, d2_exact, 0.0),
                               axis=1, keepdims=True)
                std_ref[j] = jnp.sqrt(d2nd + 1e-10) * STD_FACTOR
            else:
                d2_sel = jnp.where(onehot, jnp.inf, d2_sel)
        out_ref[0, 0] = jnp.zeros((tq, 1), jnp.float32)

    @pl.when(j >= n_self)
    def _query_phase():
        c = jax.lax.rem(j - n_self, n_self)
        qn = tgt_blk_ref[0] + noise_ref[0, 0] * std_ref[c]
        is_src = j >= n_self + n_noisy
        qv = jnp.where(is_src, src_blk_ref[0], qn)

        def top5(pt_ref, invs):
            d2_sel, d2_exact = _sel_and_exact_d2(qv, pt_ref)
            dists = []
            u_raw = jnp.zeros((tq, n), jnp.float32)
            for k in range(K):
                m = jnp.min(d2_sel, axis=1, keepdims=True)
                onehot = d2_sel <= m
                dk = jnp.sum(jnp.where(onehot, d2_exact, 0.0),
                             axis=1, keepdims=True)
                dists.append(dk)
                wk = 1.0 / (dk + 1e-8) if invs is None else invs[k]
                u_raw = u_raw + jnp.where(onehot, wk, 0.0)
                if k < K - 1:
                    d2_sel = jnp.where(onehot, jnp.inf, d2_sel)
            return dists, u_raw

        mt, u_t = top5(ptt_ref, None)
        inv = [1.0 / (m + 1e-8) for m in mt]
        norm = inv[0] + inv[1] + inv[2] + inv[3] + inv[4]
        rnorm = 1.0 / norm

        ms, u_s = top5(pts_ref, inv)

        udf_t = jnp.zeros((tq, 1), jnp.float32)
        udf_s = jnp.zeros((tq, 1), jnp.float32)
        for k in range(K):
            udf_t = udf_t + jnp.sqrt(mt[k] + 1e-10) * inv[k]
            udf_s = udf_s + jnp.sqrt(ms[k] + 1e-10) * inv[k]

        u = jnp.concatenate([u_t * rnorm, u_s * (-rnorm)],
                            axis=1).astype(jnp.bfloat16)
        wpd = (jnp.dot(u, pmh_ref[0], preferred_element_type=jnp.float32)
               + jnp.dot(u, pml_ref[0], preferred_element_type=jnp.float32))
        gd = jnp.sum(jnp.abs(wpd), axis=1, keepdims=True)
        out_ref[0, 0] = jnp.abs(udf_t - udf_s) * rnorm + gd


@jax.jit
def kernel(src, tgt):
    b, n, _ = tgt.shape
    nq = n * UP_RATIO + src.shape[1]
    n_self = n // _TQ
    n_noisy = (n * UP_RATIO) // _TQ
    n_src = src.shape[1] // _TQ
    steps = n_self + n_noisy + n_src

    tgt_t = jnp.swapaxes(tgt, 1, 2)
    src_t = jnp.swapaxes(src, 1, 2)

    pad = jnp.zeros((b, n, 5), jnp.float32)
    pm = jnp.concatenate(
        [jnp.concatenate([tgt, pad], axis=2),
         jnp.concatenate([src, pad], axis=2)], axis=1)
    pm_hi = pm.astype(jnp.bfloat16)
    pm_lo = (pm - pm_hi.astype(jnp.float32)).astype(jnp.bfloat16)

    noise = _noise_table(b, n)

    def tgt_chunk(i, j):
        c = jnp.where(j < n_self, j, jax.lax.rem(j - n_self, n_self))
        return (i, c, 0)

    def noise_chunk(i, j):
        r = jnp.clip((j - n_self) // n_self, 0, UP_RATIO - 1)
        c = jax.lax.rem(j - n_self, n_self)
        return (i, r, jnp.where(j < n_self, 0, c), 0)

    def src_chunk(i, j):
        return (i, jnp.clip(j - n_self - n_noisy, 0, n_src - 1), 0)

    err = pl.pallas_call(
        functools.partial(_fused_body, n=n, n_self=n_self, n_noisy=n_noisy),
        grid=(b, steps),
        in_specs=[
            pl.BlockSpec((1, _TQ, 3), tgt_chunk),
            pl.BlockSpec((1, _TQ, 3), src_chunk),
            pl.BlockSpec((1, 1, _TQ, 3), noise_chunk),
            pl.BlockSpec((1, 3, n), lambda i, j: (i, 0, 0)),
            pl.BlockSpec((1, 3, n), lambda i, j: (i, 0, 0)),
            pl.BlockSpec((1, 2 * n, 8), lambda i, j: (i, 0, 0)),
            pl.BlockSpec((1, 2 * n, 8), lambda i, j: (i, 0, 0)),
        ],
        out_specs=pl.BlockSpec((1, 1, _TQ, 1), lambda i, j: (i, j, 0, 0)),
        out_shape=jax.ShapeDtypeStruct((b, steps, _TQ, 1), jnp.float32),
        scratch_shapes=[pltpu.VMEM((n_self, _TQ, 1), jnp.float32)],
    )(tgt, src, noise, tgt_t, src_t, pm_hi, pm_lo)

    return jnp.sum(err[:, n_self:]) / b / nq


# speedup vs baseline: 1.0670x; 1.0527x over previous
import functools

import jax
import jax.numpy as jnp
from jax.experimental import pallas as pl
from jax.experimental.pallas import tpu as pltpu

UP_RATIO = 10
K = 5
STD_FACTOR = 3.0

_TQ = 512

_NOISE_CACHE = {}


def _noise_table(b, n):
    key = (b, n)
    if key not in _NOISE_CACHE:
        with jax.ensure_compile_time_eval():
            nz = jax.random.normal(
                jax.random.key(42), (b, n, UP_RATIO, 3), dtype=jnp.float32)
            _NOISE_CACHE[key] = jnp.swapaxes(nz, 1, 2)
    return _NOISE_CACHE[key]


def _sel_and_exact_d2(qv, pt_ref):
    qx = qv[:, 0:1]
    qy = qv[:, 1:2]
    qz = qv[:, 2:3]
    px = pt_ref[0, 0:1, :]
    py = pt_ref[0, 1:2, :]
    pz = pt_ref[0, 2:3, :]
    q2 = qx * qx + qy * qy + qz * qz
    p2 = px * px + py * py + pz * pz
    qb = qv.astype(jnp.bfloat16)
    pb2 = (-2.0 * pt_ref[0].astype(jnp.bfloat16).astype(jnp.float32)
           ).astype(jnp.bfloat16)
    qp2 = jnp.dot(qb, pb2, preferred_element_type=jnp.float32)
    d2_sel = (q2 + p2) + qp2
    dx = qx - px
    dy = qy - py
    dz = qz - pz
    d2_exact = dx * dx + dy * dy + dz * dz
    return d2_sel, d2_exact


def _fused_body(tgt_blk_ref, src_blk_ref, noise_ref, ptt_ref, pts_ref,
                pmhl_ref, out_ref, std_ref, *, n, n_self, n_noisy):
    j = pl.program_id(1)
    tq = tgt_blk_ref.shape[1]

    @pl.when(j < n_self)
    def _self_phase():
        qv = tgt_blk_ref[0]
        d2_sel, d2_exact = _sel_and_exact_d2(qv, ptt_ref)
        for rank in range(2):
            m = jnp.min(d2_sel, axis=1, keepdims=True)
            onehot = d2_sel <= m
            if rank == 1:
                d2nd = jnp.sum(jnp.where(onehot, d2_exact, 0.0),
                               axis=1, keepdims=True)
                std_ref[j] = jnp.sqrt(d2nd + 1e-10) * STD_FACTOR
            else:
                d2_sel = jnp.where(onehot, jnp.inf, d2_sel)
        out_ref[0, 0] = jnp.zeros((tq, 1), jnp.float32)

    @pl.when(j >= n_self)
    def _query_phase():
        c = jax.lax.rem(j - n_self, n_self)
        qn = tgt_blk_ref[0] + noise_ref[0, 0] * std_ref[c]
        is_src = j >= n_self + n_noisy
        qv = jnp.where(is_src, src_blk_ref[0], qn)

        def top5(pt_ref, invs):
            d2_sel, d2_exact = _sel_and_exact_d2(qv, pt_ref)
            dists = []
            u_raw = jnp.zeros((tq, n), jnp.float32)
            for k in range(K):
                m = jnp.min(d2_sel, axis=1, keepdims=True)
                onehot = d2_sel <= m
                dk = jnp.sum(jnp.where(onehot, d2_exact, 0.0),
                             axis=1, keepdims=True)
                dists.append(dk)
                wk = 1.0 / (dk + 1e-8) if invs is None else invs[k]
                u_raw = u_raw + jnp.where(onehot, wk, 0.0)
                if k < K - 1:
                    d2_sel = jnp.where(onehot, jnp.inf, d2_sel)
            return dists, u_raw

        mt, u_t = top5(ptt_ref, None)
        inv = [1.0 / (m + 1e-8) for m in mt]
        norm = inv[0] + inv[1] + inv[2] + inv[3] + inv[4]
        rnorm = 1.0 / norm

        ms, u_s = top5(pts_ref, inv)

        udf_t = jnp.zeros((tq, 1), jnp.float32)
        udf_s = jnp.zeros((tq, 1), jnp.float32)
        for k in range(K):
            udf_t = udf_t + jnp.sqrt(mt[k] + 1e-10) * inv[k]
            udf_s = udf_s + jnp.sqrt(ms[k] + 1e-10) * inv[k]

        u = jnp.concatenate([u_t * rnorm, u_s * (-rnorm)],
                            axis=1).astype(jnp.bfloat16)
        wpd16 = jnp.dot(u, pmhl_ref[0], preferred_element_type=jnp.float32)
        wpd = wpd16[:, 0:8] + wpd16[:, 8:16]
        gd = jnp.sum(jnp.abs(wpd), axis=1, keepdims=True)
        out_ref[0, 0] = jnp.abs(udf_t - udf_s) * rnorm + gd


@jax.jit
def kernel(src, tgt):
    b, n, _ = tgt.shape
    nq = n * UP_RATIO + src.shape[1]
    n_self = n // _TQ
    n_noisy = (n * UP_RATIO) // _TQ
    n_src = src.shape[1] // _TQ
    steps = n_self + n_noisy + n_src

    tgt_t = jnp.swapaxes(tgt, 1, 2)
    src_t = jnp.swapaxes(src, 1, 2)

    pad = jnp.zeros((b, n, 5), jnp.float32)
    pm = jnp.concatenate(
        [jnp.concatenate([tgt, pad], axis=2),
         jnp.concatenate([src, pad], axis=2)], axis=1)
    pm_hi = pm.astype(jnp.bfloat16)
    pm_lo = (pm - pm_hi.astype(jnp.float32)).astype(jnp.bfloat16)
    pm_hl = jnp.concatenate([pm_hi, pm_lo], axis=2)

    noise = _noise_table(b, n)

    def tgt_chunk(i, j):
        c = jnp.where(j < n_self, j, jax.lax.rem(j - n_self, n_self))
        return (i, c, 0)

    def noise_chunk(i, j):
        r = jnp.clip((j - n_self) // n_self, 0, UP_RATIO - 1)
        c = jax.lax.rem(j - n_self, n_self)
        return (i, r, jnp.where(j < n_self, 0, c), 0)

    def src_chunk(i, j):
        return (i, jnp.clip(j - n_self - n_noisy, 0, n_src - 1), 0)

    err = pl.pallas_call(
        functools.partial(_fused_body, n=n, n_self=n_self, n_noisy=n_noisy),
        grid=(b, steps),
        in_specs=[
            pl.BlockSpec((1, _TQ, 3), tgt_chunk),
            pl.BlockSpec((1, _TQ, 3), src_chunk),
            pl.BlockSpec((1, 1, _TQ, 3), noise_chunk),
            pl.BlockSpec((1, 3, n), lambda i, j: (i, 0, 0)),
            pl.BlockSpec((1, 3, n), lambda i, j: (i, 0, 0)),
            pl.BlockSpec((1, 2 * n, 16), lambda i, j: (i, 0, 0)),
        ],
        out_specs=pl.BlockSpec((1, 1, _TQ, 1), lambda i, j: (i, j, 0, 0)),
        out_shape=jax.ShapeDtypeStruct((b, steps, _TQ, 1), jnp.float32),
        scratch_shapes=[pltpu.VMEM((n_self, _TQ, 1), jnp.float32)],
    )(tgt, src, noise, tgt_t, src_t, pm_hl)

    return jnp.sum(err[:, n_self:]) / b / nq


# speedup vs baseline: 1.0673x; 1.0002x over previous
import functools

import jax
import jax.numpy as jnp
from jax.experimental import pallas as pl
from jax.experimental.pallas import tpu as pltpu

UP_RATIO = 10
K = 5
STD_FACTOR = 3.0

_TQ = 512

_NOISE_CACHE = {}


def _noise_table(b, n):
    key = (b, n)
    if key not in _NOISE_CACHE:
        with jax.ensure_compile_time_eval():
            nz = jax.random.normal(
                jax.random.key(42), (b, n, UP_RATIO, 3), dtype=jnp.float32)
            _NOISE_CACHE[key] = jnp.swapaxes(nz, 1, 2)
    return _NOISE_CACHE[key]


def _d2_pair(qv, ptc_ref, qp2, lo, n):
    qx = qv[:, 0:1]
    qy = qv[:, 1:2]
    qz = qv[:, 2:3]
    px = ptc_ref[0, 0:1, lo:lo + n]
    py = ptc_ref[0, 1:2, lo:lo + n]
    pz = ptc_ref[0, 2:3, lo:lo + n]
    q2 = qx * qx + qy * qy + qz * qz
    p2 = px * px + py * py + pz * pz
    d2_sel = (q2 + p2) + qp2
    dx = qx - px
    dy = qy - py
    dz = qz - pz
    d2_exact = dx * dx + dy * dy + dz * dz
    return d2_sel, d2_exact


def _neg2_bf16(x):
    return (-2.0 * x.astype(jnp.bfloat16).astype(jnp.float32)
            ).astype(jnp.bfloat16)


def _fused_body(tgt_blk_ref, src_blk_ref, noise_ref, ptc_ref,
                pmhl_ref, out_ref, std_ref, *, n, n_self, n_noisy):
    j = pl.program_id(1)
    tq = tgt_blk_ref.shape[1]

    @pl.when(j < n_self)
    def _self_phase():
        qv = tgt_blk_ref[0]
        qp2 = jnp.dot(qv.astype(jnp.bfloat16), _neg2_bf16(ptc_ref[0, :, 0:n]),
                      preferred_element_type=jnp.float32)
        d2_sel, d2_exact = _d2_pair(qv, ptc_ref, qp2, 0, n)
        for rank in range(2):
            m = jnp.min(d2_sel, axis=1, keepdims=True)
            onehot = d2_sel <= m
            if rank == 1:
                d2nd = jnp.sum(jnp.where(onehot, d2_exact, 0.0),
                               axis=1, keepdims=True)
                std_ref[j] = jnp.sqrt(d2nd + 1e-10) * STD_FACTOR
            else:
                d2_sel = jnp.where(onehot, jnp.inf, d2_sel)
        out_ref[0, 0] = jnp.zeros((tq, 1), jnp.float32)

    @pl.when(j >= n_self)
    def _query_phase():
        c = jax.lax.rem(j - n_self, n_self)
        qn = tgt_blk_ref[0] + noise_ref[0, 0] * std_ref[c]
        is_src = j >= n_self + n_noisy
        qv = jnp.where(is_src, src_blk_ref[0], qn)

        qp2f = jnp.dot(qv.astype(jnp.bfloat16), _neg2_bf16(ptc_ref[0]),
                       preferred_element_type=jnp.float32)

        def top5(lo, invs):
            d2_sel, d2_exact = _d2_pair(qv, ptc_ref, qp2f[:, lo:lo + n],
                                        lo, n)
            dists = []
            u_raw = jnp.zeros((tq, n), jnp.float32)
            for k in range(K):
                m = jnp.min(d2_sel, axis=1, keepdims=True)
                onehot = d2_sel <= m
                dk = jnp.sum(jnp.where(onehot, d2_exact, 0.0),
                             axis=1, keepdims=True)
                dists.append(dk)
                wk = 1.0 / (dk + 1e-8) if invs is None else invs[k]
                u_raw = u_raw + jnp.where(onehot, wk, 0.0)
                if k < K - 1:
                    d2_sel = jnp.where(onehot, jnp.inf, d2_sel)
            return dists, u_raw

        mt, u_t = top5(0, None)
        inv = [1.0 / (m + 1e-8) for m in mt]
        norm = inv[0] + inv[1] + inv[2] + inv[3] + inv[4]
        rnorm = 1.0 / norm

        ms, u_s = top5(n, inv)

        udf_t = jnp.zeros((tq, 1), jnp.float32)
        udf_s = jnp.zeros((tq, 1), jnp.float32)
        for k in range(K):
            udf_t = udf_t + jnp.sqrt(mt[k] + 1e-10) * inv[k]
            udf_s = udf_s + jnp.sqrt(ms[k] + 1e-10) * inv[k]

        u = jnp.concatenate([u_t * rnorm, u_s * (-rnorm)],
                            axis=1).astype(jnp.bfloat16)
        wpd16 = jnp.dot(u, pmhl_ref[0], preferred_element_type=jnp.float32)
        wpd = wpd16[:, 0:8] + wpd16[:, 8:16]
        gd = jnp.sum(jnp.abs(wpd), axis=1, keepdims=True)
        out_ref[0, 0] = jnp.abs(udf_t - udf_s) * rnorm + gd


@jax.jit
def kernel(src, tgt):
    b, n, _ = tgt.shape
    nq = n * UP_RATIO + src.shape[1]
    n_self = n // _TQ
    n_noisy = (n * UP_RATIO) // _TQ
    n_src = src.shape[1] // _TQ
    steps = n_self + n_noisy + n_src

    ptc = jnp.concatenate(
        [jnp.swapaxes(tgt, 1, 2), jnp.swapaxes(src, 1, 2)], axis=2)

    pad = jnp.zeros((b, n, 5), jnp.float32)
    pm = jnp.concatenate(
        [jnp.concatenate([tgt, pad], axis=2),
         jnp.concatenate([src, pad], axis=2)], axis=1)
    pm_hi = pm.astype(jnp.bfloat16)
    pm_lo = (pm - pm_hi.astype(jnp.float32)).astype(jnp.bfloat16)
    pm_hl = jnp.concatenate([pm_hi, pm_lo], axis=2)

    noise = _noise_table(b, n)

    def tgt_chunk(i, j):
        c = jnp.where(j < n_self, j, jax.lax.rem(j - n_self, n_self))
        return (i, c, 0)

    def noise_chunk(i, j):
        r = jnp.clip((j - n_self) // n_self, 0, UP_RATIO - 1)
        c = jax.lax.rem(j - n_self, n_self)
        return (i, r, jnp.where(j < n_self, 0, c), 0)

    def src_chunk(i, j):
        return (i, jnp.clip(j - n_self - n_noisy, 0, n_src - 1), 0)

    err = pl.pallas_call(
        functools.partial(_fused_body, n=n, n_self=n_self, n_noisy=n_noisy),
        grid=(b, steps),
        in_specs=[
            pl.BlockSpec((1, _TQ, 3), tgt_chunk),
            pl.BlockSpec((1, _TQ, 3), src_chunk),
            pl.BlockSpec((1, 1, _TQ, 3), noise_chunk),
            pl.BlockSpec((1, 3, 2 * n), lambda i, j: (i, 0, 0)),
            pl.BlockSpec((1, 2 * n, 16), lambda i, j: (i, 0, 0)),
        ],
        out_specs=pl.BlockSpec((1, 1, _TQ, 1), lambda i, j: (i, j, 0, 0)),
        out_shape=jax.ShapeDtypeStruct((b, steps, _TQ, 1), jnp.float32),
        scratch_shapes=[pltpu.VMEM((n_self, _TQ, 1), jnp.float32)],
    )(tgt, src, noise, ptc, pm_hl)

    return jnp.sum(err[:, n_self:]) / b / nq
